# baseline (speedup 1.0000x reference)
import functools

import jax
import jax.numpy as jnp
from jax import lax
from jax.experimental import pallas as pl
from jax.experimental.pallas import tpu as pltpu
from jax.experimental.pallas import tpu_sc as plsc

NC = 2
NS = 16
NW = NC * NS


@functools.partial(jax.jit, static_argnames=("b", "s", "d"))
def _gather_rows(ids, table, b, s, d):
    rows_per_w = (b * s) // NW
    w_per_b = s // rows_per_w
    chunk = 8
    nbuf = 4
    n_chunks = rows_per_w // chunk
    n_waves = n_chunks // nbuf

    mesh = plsc.VectorSubcoreMesh(core_axis_name="c", subcore_axis_name="s")

    @functools.partial(
        pl.kernel,
        out_type=jax.ShapeDtypeStruct((b, s, d), jnp.float32),
        mesh=mesh,
        scratch_types=[
            pltpu.VMEM((rows_per_w,), jnp.int32),
            *[pltpu.VMEM((chunk, d), jnp.float32) for _ in range(nbuf)],
            *[pltpu.SemaphoreType.DMA for _ in range(2 * nbuf)],
        ],
    )
    def k(ids_hbm, table_hbm, out_hbm, idx_v, *scr):
        bufs = scr[:nbuf]
        gsems = scr[nbuf : 2 * nbuf]
        ssems = scr[2 * nbuf :]
        wid = lax.axis_index("s") * NC + lax.axis_index("c")
        b_idx = wid // w_per_b
        col0 = (wid % w_per_b) * rows_per_w
        pltpu.sync_copy(ids_hbm.at[b_idx, pl.ds(col0, rows_per_w)], idx_v)

        def fire_gather(g, p):
            return pltpu.async_copy(
                table_hbm.at[idx_v.at[pl.ds(g * chunk, chunk)]], bufs[p], gsems[p]
            )

        def fire_store(g, p):
            return pltpu.async_copy(
                bufs[p], out_hbm.at[b_idx, pl.ds(col0 + g * chunk, chunk)], ssems[p]
            )

        for p in range(nbuf):
            fire_gather(p, p)

        def wave(w, carry):
            for p in range(nbuf):
                g = w * nbuf + p
                pltpu.make_async_copy(
                    table_hbm.at[pl.ds(0, chunk)], bufs[p], gsems[p]
                ).wait()
                fire_store(g, p)
            for p in range(nbuf):
                g = w * nbuf + p
                pltpu.make_async_copy(
                    bufs[p],
                    out_hbm.at[b_idx, pl.ds(col0 + g * chunk, chunk)],
                    ssems[p],
                ).wait()

                @pl.when(w + 1 < n_waves)
                def _():
                    fire_gather(g + nbuf, p)

            return carry

        lax.fori_loop(0, n_waves, wave, 0)

    return k(ids, table)


def kernel(input_ids, table):
    b, s = input_ids.shape
    v, d = table.shape
    if input_ids.dtype != jnp.int32:
        input_ids = input_ids.astype(jnp.int32)
    return _gather_rows(input_ids, table, b, s, d)


# speedup vs baseline: 1.0374x; 1.0374x over previous
import functools

import jax
import jax.numpy as jnp
from jax import lax
from jax.experimental import pallas as pl
from jax.experimental.pallas import tpu as pltpu
from jax.experimental.pallas import tpu_sc as plsc

NC = 2
NS = 16
NW = NC * NS


@functools.partial(jax.jit, static_argnames=("b", "s", "d"))
def _gather_rows(ids, table, b, s, d):
    rows_per_w = (b * s) // NW
    w_per_b = s // rows_per_w
    chunk = 8
    nbuf = 6
    n_chunks = rows_per_w // chunk

    mesh = plsc.VectorSubcoreMesh(core_axis_name="c", subcore_axis_name="s")

    @functools.partial(
        pl.kernel,
        out_type=jax.ShapeDtypeStruct((b, s, d), jnp.float32),
        mesh=mesh,
        scratch_types=[
            pltpu.VMEM((rows_per_w,), jnp.int32),
            *[pltpu.VMEM((chunk, d), jnp.float32) for _ in range(nbuf)],
            *[pltpu.SemaphoreType.DMA for _ in range(2 * nbuf)],
        ],
    )
    def k(ids_hbm, table_hbm, out_hbm, idx_v, *scr):
        bufs = scr[:nbuf]
        gsems = scr[nbuf : 2 * nbuf]
        ssems = scr[2 * nbuf :]
        wid = lax.axis_index("s") * NC + lax.axis_index("c")
        b_idx = wid // w_per_b
        col0 = (wid % w_per_b) * rows_per_w
        base = wid * rows_per_w
        def fire_gather(g):
            p = g % nbuf
            return pltpu.async_copy(
                table_hbm.at[idx_v.at[pl.ds(g * chunk, chunk)]], bufs[p], gsems[p]
            )

        head = (nbuf - 1) * chunk
        pltpu.sync_copy(ids_hbm.at[pl.ds(base, head)], idx_v.at[pl.ds(0, head)])
        gathers = {}
        stores = {}
        for g in range(min(nbuf - 1, n_chunks)):
            gathers[g] = fire_gather(g)
        pltpu.sync_copy(
            ids_hbm.at[pl.ds(base + head, rows_per_w - head)],
            idx_v.at[pl.ds(head, rows_per_w - head)],
        )
        for g in range(n_chunks):
            p = g % nbuf
            gathers[g].wait()
            stores[g] = pltpu.async_copy(
                bufs[p], out_hbm.at[b_idx, pl.ds(col0 + g * chunk, chunk)], ssems[p]
            )
            nxt = g + nbuf - 1
            if nxt < n_chunks:
                if g >= 1:
                    stores[g - 1].wait()
                gathers[nxt] = fire_gather(nxt)
        for g in range(max(0, n_chunks - nbuf), n_chunks):
            stores[g].wait()

    return k(ids.reshape(b * s), table)


def kernel(input_ids, table):
    b, s = input_ids.shape
    v, d = table.shape
    if input_ids.dtype != jnp.int32:
        input_ids = input_ids.astype(jnp.int32)
    return _gather_rows(input_ids, table, b, s, d)
